# baseline (speedup 1.0000x reference)
import functools

import jax
import jax.numpy as jnp
from jax import lax
from jax.experimental import pallas as pl
from jax.experimental.pallas import tpu as pltpu
from jax.experimental.pallas import tpu_sc as plsc

N = 10000
E = 320000
D = 128

_NC = 2
_NS = 16
_NW = _NC * _NS


def _dot(a, b):
    return jnp.dot(a, b, preferred_element_type=jnp.float32)


def _row_sum(x):
    acc = x[:, 0:8]
    for i in range(1, 16):
        acc = acc + x[:, i * 8:(i + 1) * 8]
    while acc.shape[1] > 1:
        s = acc.shape[1] // 2
        acc = acc[:, :s] + acc[:, s:]
    return acc


def _mlp_body(n_in, groups, has_ln):
    def body(*refs):
        ins = refs[:n_in]
        w1 = refs[n_in]
        idx = n_in + 1
        b1 = refs[idx]
        idx += 1
        slabs = []
        for grp in groups:
            a = ins[grp[0]][...]
            for k in grp[1:]:
                a = a + ins[k][...]
            slabs.append(a)
        a = slabs[0] if len(slabs) == 1 else jnp.concatenate(slabs, axis=1)
        h = jnp.maximum(_dot(a, w1[...]) + b1[...], 0.0)
        for li in range(3):
            w = refs[idx][...]
            b = refs[idx + 1][...]
            idx += 2
            h = _dot(h, w) + b
            if li < 2:
                h = jnp.maximum(h, 0.0)
        if has_ln:
            g = refs[idx][...]
            bb = refs[idx + 1][...]
            idx += 2
            mu = _row_sum(h) * (1.0 / 128.0)
            d = h - mu
            var = _row_sum(d * d) * (1.0 / 128.0)
            h = d / jnp.sqrt(var + 1e-5) * g + bb
        refs[-1][...] = h
    return body


def _const_spec(shape):
    nd = len(shape)
    return pl.BlockSpec(shape, lambda i: (0,) * nd)


def _mlp(inputs, in_maps, w1, b1, rest, ln, block_rows, n_rows,
         groups=None):
    n_in = len(inputs)
    if groups is None:
        groups = [[k] for k in range(n_in)]
    grid = (n_rows // block_rows,)
    in_specs = [pl.BlockSpec((block_rows, D), m) for m in in_maps]
    args = list(inputs)
    args.append(w1)
    in_specs.append(_const_spec(w1.shape))
    args.append(b1.reshape(1, D))
    in_specs.append(_const_spec((1, D)))
    for (w, b) in rest:
        args.append(w)
        in_specs.append(_const_spec(w.shape))
        args.append(b.reshape(1, D))
        in_specs.append(_const_spec((1, D)))
    if ln is not None:
        for v in ln:
            args.append(v.reshape(1, D))
            in_specs.append(_const_spec((1, D)))
    return pl.pallas_call(
        _mlp_body(n_in, groups, ln is not None),
        grid=grid,
        in_specs=in_specs,
        out_specs=pl.BlockSpec((block_rows, D), lambda i: (i, 0)),
        out_shape=jax.ShapeDtypeStruct((n_rows, D), jnp.float32),
    )(*args)


def _pad_w(w):
    r, c = w.shape
    rp = (-r) % D
    cp = (-c) % D
    if rp or cp:
        w = jnp.pad(w, ((0, rp), (0, cp)))
    return w


def _pad_b(b):
    cp = (-b.shape[0]) % D
    if cp:
        b = jnp.pad(b, ((0, cp),))
    return b


_G_B = 2 * E
_G_PW = _G_B // _NW
_G_CH = 80
_G_NCH = _G_PW // _G_CH

def _sc_mesh():
    return plsc.VectorSubcoreMesh(core_axis_name="c", subcore_axis_name="s",
                                  num_cores=_NC, num_subcores=_NS)


@functools.cache
def _gather_kernel():
    @functools.partial(
        pl.kernel,
        out_type=jax.ShapeDtypeStruct((_G_B, D), jnp.float32),
        mesh=_sc_mesh(),
        scratch_types=[
            pltpu.VMEM((_G_CH,), jnp.int32),
            pltpu.VMEM((_G_CH, D), jnp.float32),
            pltpu.SemaphoreType.DMA,
        ],
    )
    def _gather(table_hbm, idx_hbm, out_hbm, idx_v, rows_v, sem):
        wid = lax.axis_index("s") * _NC + lax.axis_index("c")
        base = wid * _G_PW

        def step(g, _):
            off = base + g * _G_CH
            pltpu.sync_copy(idx_hbm.at[pl.ds(off, _G_CH)], idx_v)
            pltpu.async_copy(table_hbm.at[idx_v], rows_v, sem).wait()
            pltpu.sync_copy(rows_v, out_hbm.at[pl.ds(off, _G_CH)])
            return 0

        lax.fori_loop(0, _G_NCH, step, 0)

    return _gather


_W_ROWS = 312
_RC = 4000
_NSEG = E // _RC
_SEGCAP = 4096
_S_CH = 128
_NA = N + 16


@functools.cache
def _prep_kernel():
    @functools.partial(
        pl.kernel,
        out_type=(
            jax.ShapeDtypeStruct((_NW * _NSEG * _SEGCAP,), jnp.int32),
            jax.ShapeDtypeStruct((_NW * _NSEG * _SEGCAP,), jnp.int32),
        ),
        mesh=_sc_mesh(),
        compiler_params=pltpu.CompilerParams(needs_layout_passes=False),
        scratch_types=[
            pltpu.VMEM((_RC,), jnp.int32),
            pltpu.VMEM((_SEGCAP + 160,), jnp.int32),
            pltpu.VMEM((_SEGCAP + 160,), jnp.int32),
        ],
    )
    def _prep(recv_hbm, eid_hbm, tgt_hbm, recv_v, ebuf, tbuf):
        cid = lax.axis_index("c")
        sid = lax.axis_index("s")
        wid = sid * _NC + cid
        lanes = lax.iota(jnp.int32, 16)
        zero_v = lanes * 0
        lo_v = zero_v + wid * _W_ROWS
        hi_v = jnp.where(zero_v + wid == _NW - 1, N, lo_v + _W_ROWS)
        trash = zero_v + (N + sid)
        dummy_e = zero_v + wid * 16 + lanes
        shift_idx = [jnp.maximum(lanes - d, 0) for d in (1, 2, 4, 8)]
        shift_msk = [lanes >= d for d in (1, 2, 4, 8)]

        def vgather(s, ix):
            return lax.gather(
                s, ix[:, None],
                dimension_numbers=lax.GatherDimensionNumbers(
                    offset_dims=(), collapsed_slice_dims=(0,),
                    start_index_map=(0,)),
                slice_sizes=(1,),
                mode=lax.GatherScatterMode.PROMISE_IN_BOUNDS)

        def prefix(m32):
            s = m32
            for ix, mk in zip(shift_idx, shift_msk):
                s = s + jnp.where(mk, vgather(s, ix), 0)
            return s

        splat_last = jnp.full((16,), 15, jnp.int32)

        def seg(oc, _):
            pltpu.sync_copy(recv_hbm.at[pl.ds(oc * _RC, _RC)], recv_v)
            ids0 = zero_v + oc * _RC + lanes

            def step(i, carry):
                off_v, ids = carry
                r = recv_v[pl.ds(i * 16, 16)]
                mask = (r >= lo_v) & (r < hi_v)
                m32 = jnp.where(mask, 1, 0)
                inc = prefix(m32)
                pos = off_v + inc - m32
                plsc.store_scatter(ebuf, [pos], ids, mask=mask)
                plsc.store_scatter(tbuf, [pos], r, mask=mask)
                tot = vgather(inc, splat_last)
                return (off_v + tot, ids + 16)

            off_v, _ids = lax.fori_loop(0, _RC // 16, step,
                                        (zero_v, ids0))

            full = zero_v == 0
            for k in range(10):
                ppos = off_v + lanes + k * 16
                plsc.store_scatter(ebuf, [ppos], dummy_e, mask=full)
                plsc.store_scatter(tbuf, [ppos], trash, mask=full)

            seg_off = (wid * _NSEG + oc) * _SEGCAP
            pltpu.sync_copy(ebuf.at[pl.ds(0, _SEGCAP)],
                            eid_hbm.at[pl.ds(seg_off, _SEGCAP)])
            pltpu.sync_copy(tbuf.at[pl.ds(0, _SEGCAP)],
                            tgt_hbm.at[pl.ds(seg_off, _SEGCAP)])
            return 0

        lax.fori_loop(0, _NSEG, seg, 0)

    return _prep


@functools.cache
def _scatter_kernel():
    @functools.partial(
        pl.kernel,
        out_type=jax.ShapeDtypeStruct((N, D), jnp.float32),
        mesh=_sc_mesh(),
        compiler_params=pltpu.CompilerParams(needs_layout_passes=False),
        scratch_types=[
            pltpu.VMEM((_S_CH,), jnp.int32),
            pltpu.VMEM((_S_CH,), jnp.int32),
            pltpu.VMEM((_S_CH, D), jnp.float32),
            pltpu.VMEM_SHARED((_NA, D), jnp.float32),
            pltpu.SemaphoreType.DMA,
        ],
    )
    def _scatter_add(edge_hbm, eid_hbm, tgt_hbm, zeros_hbm, out_hbm,
                     eid_v, tgt_v, rows_v, acc, sem):
        cid = lax.axis_index("c")
        sid = lax.axis_index("s")
        wid = sid * _NC + cid
        pltpu.sync_copy(zeros_hbm.at[pl.ds(sid * 624, 624)],
                        acc.at[pl.ds(sid * 624, 624)])

        @pl.when(sid == 0)
        def _():
            pltpu.sync_copy(zeros_hbm.at[pl.ds(9984, _NA - 9984)],
                            acc.at[pl.ds(9984, _NA - 9984)])

        plsc.subcore_barrier()
        nsent = jnp.full((16,), N, jnp.int32)

        def seg(oc, _):
            seg_off = (wid * _NSEG + oc) * _SEGCAP

            def cond(carry):
                return carry[1]

            def chunk(carry):
                j, _cont = carry
                off = seg_off + j * _S_CH
                pltpu.sync_copy(eid_hbm.at[pl.ds(off, _S_CH)], eid_v)
                pltpu.sync_copy(tgt_hbm.at[pl.ds(off, _S_CH)], tgt_v)
                pltpu.async_copy(edge_hbm.at[eid_v], rows_v, sem).wait()
                pltpu.sync_copy(rows_v, acc.at[tgt_v], add=True)
                pl.delay(1024)
                tail = tgt_v[pl.ds(_S_CH - 16, 16)]
                more = jnp.logical_not(jnp.any(tail >= nsent))
                return (j + 1, more)

            lax.while_loop(cond, chunk, (jnp.int32(0), True))
            return 0

        lax.fori_loop(0, _NSEG, seg, 0)
        plsc.subcore_barrier()

        @pl.when(wid < _NW - 1)
        def _():
            pltpu.sync_copy(acc.at[pl.ds(wid * _W_ROWS, _W_ROWS)],
                            out_hbm.at[pl.ds(wid * _W_ROWS, _W_ROWS)])

        @pl.when(wid == _NW - 1)
        def _():
            lastb = (_NW - 1) * _W_ROWS
            pltpu.sync_copy(acc.at[pl.ds(lastb, N - lastb)],
                            out_hbm.at[pl.ds(lastb, N - lastb)])

    return _scatter_add


_BN = 1000
_BE = 2000


def _mlp_args(ps):
    (w1, b1) = ps[0]
    rest = [(w, b) for (w, b) in ps[1:]]
    return w1, b1, rest


def kernel(x, edge_attr, params, edge_index):
    zeros_nd = jnp.zeros((_NA, D), jnp.float32)
    row_map = lambda i: (i, 0)

    w1, b1, rest = _mlp_args(params['enc_node']['mlp'])
    xp = jnp.pad(x, ((0, 0), (0, D - x.shape[1])))
    node = _mlp([xp], [row_map], _pad_w(w1), b1, rest,
                params['enc_node']['ln'], _BN, N)

    w1, b1, rest = _mlp_args(params['enc_edge']['mlp'])
    eap = jnp.pad(edge_attr, ((0, 0), (0, D - edge_attr.shape[1])))
    edge = _mlp([eap], [row_map], _pad_w(w1), b1, rest,
                params['enc_edge']['ln'], _BE, E)

    send = edge_index[0]
    recv = edge_index[1]
    idx2 = jnp.concatenate([send, recv])
    eids, tgts = _prep_kernel()(recv)

    nb = E // _BE

    for layer in params['mp']:
        gath = _gather_kernel()(node, idx2)

        w1, b1, rest = _mlp_args(layer['edge']['mlp'])
        edge = _mlp(
            [gath, gath, edge],
            [row_map, lambda i: (i + nb, 0), row_map],
            w1, b1, rest, layer['edge']['ln'], _BE, E,
            groups=[[0], [1], [2]])

        agg = _scatter_kernel()(edge, eids, tgts, zeros_nd)

        w1, b1, rest = _mlp_args(layer['node']['mlp'])
        node = _mlp(
            [node, agg],
            [row_map, row_map],
            w1, b1, rest, layer['node']['ln'], _BN, N,
            groups=[[0], [1]])

    w1, b1, rest = _mlp_args(params['dec']['mlp'])
    (w4, b4) = rest[-1]
    rest = rest[:-1] + [(_pad_w(w4), _pad_b(b4))]
    out = _mlp([node], [row_map], w1, b1, rest, None, _BN, N)
    return out[:, 1:3]
